# baseline (speedup 1.0000x reference)
import functools

import jax
import jax.numpy as jnp
from jax import lax
from jax.experimental import pallas as pl
from jax.experimental.pallas import tpu as pltpu
from jax.experimental.pallas import tpu_sc as plsc

_N = 10000
_D = 128
_E = 320000

_NC = 2
_NS = 16
_NW = _NC * _NS
_EPW = _E // _NW
_CH = 125
_EPWP = 10500
_NCH = _EPWP // _CH
_CNCH = _EPW // _CH
_RPT = 624
_ZR = 128


def _per_tile(s, fn):
    @pl.when(s == _NS - 1)
    def _():
        fn((_NS - 1) * _RPT, [120] * 5 + [40])
    @pl.when(s < _NS - 1)
    def _():
        fn(s * _RPT, [120] * 5 + [24])


_NB = 2


def _sc_body(h_hbm, e_hbm, out_hbm, ring, rows_v, acc_sh, *sems):
    isems, gsems, wsems = sems[:6], sems[6:9], sems[9:]
    c = lax.axis_index("c")
    s = lax.axis_index("s")
    wid = s * _NC + c

    z16 = jnp.zeros((16,), jnp.float32)

    def _zrow(i, _):
        for j in range(_D // 16):
            rows_v[0, i, pl.ds(j * 16, 16)] = z16
        return 0
    lax.fori_loop(0, _CH, _zrow, 0)

    def _zero_acc(base, sizes):
        descs = []
        off = 0
        for sz in sizes:
            d = pltpu.make_async_copy(rows_v.at[0].at[pl.ds(0, sz)],
                                      acc_sh.at[pl.ds(base + off, sz)],
                                      wsems[0])
            d.start()
            descs.append(d)
            off += sz
        for d in descs:
            d.wait()
    _per_tile(s, _zero_acc)

    plsc.subcore_barrier()

    for t in range(4):
        pltpu.async_copy(e_hbm.at[wid, t], ring.at[t], isems[t])
    for b in range(2):
        pltpu.make_async_copy(e_hbm.at[wid, b], ring.at[b], isems[b]).wait()
        pltpu.async_copy(h_hbm.at[ring.at[b, 0]], rows_v.at[b], gsems[b])

    def _group(g, _):
        for u in range(12):
            j = g * 12 + u
            r, i6 = u % 3, u % 6
            r2, n6 = (u + 2) % 3, (u + 2) % 6
            i4 = (u + 4) % 6
            pltpu.make_async_copy(h_hbm.at[ring.at[i6, 0]],
                                  rows_v.at[r], gsems[r]).wait()
            @pl.when(j + 2 < _NCH)
            def _():
                pltpu.make_async_copy(e_hbm.at[wid, j + 2], ring.at[n6],
                                      isems[n6]).wait()
                pltpu.async_copy(h_hbm.at[ring.at[n6, 0]], rows_v.at[r2],
                                 gsems[r2])
            pltpu.sync_copy(rows_v.at[r], acc_sh.at[ring.at[i6, 1]],
                            add=True)
            @pl.when(j + 4 < _NCH)
            def _():
                pltpu.async_copy(e_hbm.at[wid, j + 4], ring.at[i4],
                                 isems[i4])
        return 0
    lax.fori_loop(0, _NCH // 12, _group, 0)

    plsc.subcore_barrier()

    def _write_acc(base, sizes):
        bufs = [rows_v.at[0], rows_v.at[1]]
        descs = []
        off = 0
        for k, sz in enumerate(sizes):
            b = k % 2
            if k >= 2:
                descs[k - 2].wait()
            pltpu.sync_copy(acc_sh.at[pl.ds(base + off, sz)],
                            bufs[b].at[pl.ds(0, sz)])
            d = pltpu.make_async_copy(bufs[b].at[pl.ds(0, sz)],
                                      out_hbm.at[c, pl.ds(base + off, sz)],
                                      wsems[b])
            d.start()
            descs.append(d)
            off += sz
        descs[-2].wait()
        descs[-1].wait()
    _per_tile(s, _write_acc)


def _cnt_body(dst_hbm, cnt_hbm, dst_all, ones_v, cbuf, cnt_sh, csem):
    c = lax.axis_index("c")
    s = lax.axis_index("s")
    wid = s * _NC + c

    z16 = jnp.zeros((16,), jnp.float32)

    def _crow(i, _):
        cbuf[i, :] = z16
        return 0
    lax.fori_loop(0, _RPT + 16, _crow, 0)

    def _zero_cnt(base, sizes):
        n = sum(sizes)
        pltpu.sync_copy(cbuf.at[pl.ds(0, n)], cnt_sh.at[pl.ds(base, n)])
    _per_tile(s, _zero_cnt)

    o16 = jnp.ones((16,), jnp.float32)
    def _orow(i, _):
        ones_v[i, :] = o16
        return 0
    lax.fori_loop(0, _CH, _orow, 0)

    pltpu.sync_copy(dst_hbm.at[wid], dst_all)

    plsc.subcore_barrier()

    def _ebody(j, _):
        pltpu.make_async_copy(ones_v, cnt_sh.at[dst_all.at[j]],
                              csem).start(add=True)
        return 0
    lax.fori_loop(0, _CNCH, _ebody, 0)

    def _edrain(j, _):
        pltpu.make_async_copy(ones_v, cnt_sh.at[dst_all.at[j]], csem).wait()
        return 0
    lax.fori_loop(0, _CNCH, _edrain, 0)

    plsc.subcore_barrier()

    def _write_cnt(base, sizes):
        n = sum(sizes)
        pltpu.sync_copy(cnt_sh.at[pl.ds(base, n)], cbuf.at[pl.ds(0, n)])
        pltpu.sync_copy(cbuf.at[pl.ds(0, n)], cnt_hbm.at[c, pl.ds(base, n)])
    _per_tile(s, _write_cnt)


def _make_sc():
    mesh = plsc.VectorSubcoreMesh(core_axis_name="c", subcore_axis_name="s")
    return pl.kernel(
        _sc_body,
        mesh=mesh,
        out_type=[jax.ShapeDtypeStruct((_NC, _N, _D), jnp.float32)],
        scratch_types=[
            pltpu.VMEM((6, 2, _CH), jnp.int32),
            pltpu.VMEM((3, _CH, _D), jnp.float32),
            pltpu.VMEM_SHARED((_N + _NS, _D), jnp.float32),
        ] + [pltpu.SemaphoreType.DMA] * 11,
        compiler_params=pltpu.CompilerParams(use_tc_tiling_on_sc=False),
    )


def _make_cnt():
    mesh = plsc.VectorSubcoreMesh(core_axis_name="c", subcore_axis_name="s")
    return pl.kernel(
        _cnt_body,
        mesh=mesh,
        out_type=[jax.ShapeDtypeStruct((_NC, _N, 16), jnp.float32)],
        scratch_types=[
            pltpu.VMEM((_CNCH, _CH), jnp.int32),
            pltpu.VMEM((_CH, 16), jnp.float32),
            pltpu.VMEM((_RPT + 16, 16), jnp.float32),
            pltpu.VMEM_SHARED((_N, 16), jnp.float32),
            pltpu.SemaphoreType.DMA,
        ],
        compiler_params=pltpu.CompilerParams(use_tc_tiling_on_sc=False),
    )


def _tc_body(relu, res, p_ref, c_ref, h_ref, wl_ref, wr_ref, b_ref, o_ref):
    p = p_ref[...]
    agg = p[0] + p[1]
    cc = c_ref[...]
    cnt = (cc[0] + cc[1])[:, 0:1]
    mean = agg / jnp.maximum(cnt, 1.0)
    hh = h_ref[...]
    dn = (((1,), (1,)), ((), ()))
    out = (lax.dot_general(mean, wl_ref[...], dn,
                           preferred_element_type=jnp.float32,
                           precision=lax.Precision.HIGHEST)
           + lax.dot_general(hh, wr_ref[...], dn,
                             preferred_element_type=jnp.float32,
                             precision=lax.Precision.HIGHEST)
           + b_ref[...])
    if res:
        out = out + hh
    if relu:
        out = jnp.maximum(out, 0.0)
    o_ref[...] = out


def _tc_call(relu, res, part, cntp, h, wl, wr, b):
    B = 1000
    return pl.pallas_call(
        functools.partial(_tc_body, relu, res),
        grid=(_N // B,),
        in_specs=[
            pl.BlockSpec((_NC, B, _D), lambda i: (0, i, 0)),
            pl.BlockSpec((_NC, B, 16), lambda i: (0, i, 0)),
            pl.BlockSpec((B, _D), lambda i: (i, 0)),
            pl.BlockSpec((_D, _D), lambda i: (0, 0)),
            pl.BlockSpec((_D, _D), lambda i: (0, 0)),
            pl.BlockSpec((1, _D), lambda i: (0, 0)),
        ],
        out_specs=pl.BlockSpec((B, _D), lambda i: (i, 0)),
        out_shape=jax.ShapeDtypeStruct((_N, _D), jnp.float32),
    )(part, cntp, h, wl, wr, b)


def kernel(x, edge_index, Wl0, Wr0, b0, Wl1, Wr1, b1, Wl2, Wr2, b2):
    pad = _EPWP - _EPW
    srcp = jnp.concatenate(
        [edge_index[0].reshape(_NW, _EPW),
         jnp.zeros((_NW, pad), jnp.int32)], axis=1).reshape(_NW, _NCH, _CH)
    sink = _N + jnp.arange(_NW, dtype=jnp.int32) // _NC
    dstp = jnp.concatenate(
        [edge_index[1].reshape(_NW, _EPW),
         jnp.broadcast_to(sink[:, None], (_NW, pad))],
        axis=1).reshape(_NW, _NCH, _CH)
    e3 = jnp.stack([srcp, dstp], axis=2)
    dstc = edge_index[1].reshape(_NW, _CNCH, _CH)

    sc = _make_sc()
    (cntp,) = _make_cnt()(dstc)
    (part0,) = sc(x, e3)
    h1 = _tc_call(True, False, part0, cntp, x, Wl0, Wr0, b0.reshape(1, _D))
    (part1,) = sc(h1, e3)
    h2 = _tc_call(True, True, part1, cntp, h1, Wl1, Wr1, b1.reshape(1, _D))
    (part2,) = sc(h2, e3)
    return _tc_call(False, False, part2, cntp, h2, Wl2, Wr2, b2.reshape(1, _D))


# speedup vs baseline: 5.2467x; 5.2467x over previous
import functools

import jax
import jax.numpy as jnp
from jax import lax
from jax.experimental import pallas as pl
from jax.experimental.pallas import tpu as pltpu
from jax.experimental.pallas import tpu_sc as plsc

_N = 10000
_D = 128
_E = 320000

_NC = 2
_NS = 16
_NW = _NC * _NS
_EPW = _E // _NW
_CH = 125
_NCH = _EPW // _CH
_RPT = 624
_ZR = 128


def _per_tile(s, fn):
    @pl.when(s == _NS - 1)
    def _():
        fn((_NS - 1) * _RPT, [120] * 5 + [40])
    @pl.when(s < _NS - 1)
    def _():
        fn(s * _RPT, [120] * 5 + [24])


_NB = 2


def _sc_body(h_hbm, e_hbm, out_hbm, ring, rows_v, zbuf, acc_sh, *sems):
    isems, gsems, wsems = sems[:4], sems[4:6], sems[6:]
    c = lax.axis_index("c")
    s = lax.axis_index("s")
    wid = s * _NC + c

    z16 = jnp.zeros((16,), jnp.float32)

    def _zrow(i, _):
        for j in range(_D // 16):
            zbuf[i, pl.ds(j * 16, 16)] = z16
        return 0
    lax.fori_loop(0, _ZR, _zrow, 0)

    def _zero_acc(base, sizes):
        descs = []
        off = 0
        for sz in sizes:
            d = pltpu.make_async_copy(zbuf.at[pl.ds(0, sz)],
                                      acc_sh.at[pl.ds(base + off, sz)],
                                      wsems[0])
            d.start()
            descs.append(d)
            off += sz
        for d in descs:
            d.wait()
    _per_tile(s, _zero_acc)

    plsc.subcore_barrier()

    for t in range(4):
        pltpu.async_copy(e_hbm.at[wid, t], ring.at[t], isems[t])
    for b in range(2):
        pltpu.make_async_copy(e_hbm.at[wid, b], ring.at[b], isems[b]).wait()
        pltpu.async_copy(h_hbm.at[ring.at[b, 0]], rows_v.at[b], gsems[b])

    def _group(g, _):
        for u in range(4):
            j = g * 4 + u
            s2, s4 = u % 2, u
            n4, i4 = (u + 2) % 4, u
            pltpu.make_async_copy(h_hbm.at[ring.at[s4, 0]],
                                  rows_v.at[s2], gsems[s2]).wait()
            pltpu.sync_copy(rows_v.at[s2], acc_sh.at[ring.at[s4, 1]],
                            add=True)
            @pl.when(j + 2 < _NCH)
            def _():
                pltpu.make_async_copy(e_hbm.at[wid, j + 2], ring.at[n4],
                                      isems[n4]).wait()
                pltpu.async_copy(h_hbm.at[ring.at[n4, 0]], rows_v.at[s2],
                                 gsems[s2])
            @pl.when(j + 4 < _NCH)
            def _():
                pltpu.async_copy(e_hbm.at[wid, j + 4], ring.at[i4],
                                 isems[i4])
        return 0
    lax.fori_loop(0, _NCH // 4, _group, 0)

    plsc.subcore_barrier()

    def _write_acc(base, sizes):
        bufs = [zbuf, rows_v.at[0]]
        descs = []
        off = 0
        for k, sz in enumerate(sizes):
            b = k % 2
            if k >= 2:
                descs[k - 2].wait()
            pltpu.sync_copy(acc_sh.at[pl.ds(base + off, sz)],
                            bufs[b].at[pl.ds(0, sz)])
            d = pltpu.make_async_copy(bufs[b].at[pl.ds(0, sz)],
                                      out_hbm.at[c, pl.ds(base + off, sz)],
                                      wsems[b])
            d.start()
            descs.append(d)
            off += sz
        descs[-2].wait()
        descs[-1].wait()
    _per_tile(s, _write_acc)


def _cnt_body(dst_hbm, cnt_hbm, dst_all, ones_v, cbuf, cnt_sh, csem):
    c = lax.axis_index("c")
    s = lax.axis_index("s")
    wid = s * _NC + c

    z16 = jnp.zeros((16,), jnp.float32)

    def _crow(i, _):
        cbuf[i, :] = z16
        return 0
    lax.fori_loop(0, _RPT + 16, _crow, 0)

    def _zero_cnt(base, sizes):
        n = sum(sizes)
        pltpu.sync_copy(cbuf.at[pl.ds(0, n)], cnt_sh.at[pl.ds(base, n)])
    _per_tile(s, _zero_cnt)

    o16 = jnp.ones((16,), jnp.float32)
    def _orow(i, _):
        ones_v[i, :] = o16
        return 0
    lax.fori_loop(0, _CH, _orow, 0)

    pltpu.sync_copy(dst_hbm.at[wid], dst_all)

    plsc.subcore_barrier()

    def _ebody(j, _):
        pltpu.make_async_copy(ones_v, cnt_sh.at[dst_all.at[j]],
                              csem).start(add=True)
        return 0
    lax.fori_loop(0, _NCH, _ebody, 0)

    def _edrain(j, _):
        pltpu.make_async_copy(ones_v, cnt_sh.at[dst_all.at[j]], csem).wait()
        return 0
    lax.fori_loop(0, _NCH, _edrain, 0)

    plsc.subcore_barrier()

    def _write_cnt(base, sizes):
        n = sum(sizes)
        pltpu.sync_copy(cnt_sh.at[pl.ds(base, n)], cbuf.at[pl.ds(0, n)])
        pltpu.sync_copy(cbuf.at[pl.ds(0, n)], cnt_hbm.at[c, pl.ds(base, n)])
    _per_tile(s, _write_cnt)


def _make_sc():
    mesh = plsc.VectorSubcoreMesh(core_axis_name="c", subcore_axis_name="s")
    return pl.kernel(
        _sc_body,
        mesh=mesh,
        out_type=[jax.ShapeDtypeStruct((_NC, _N, _D), jnp.float32)],
        scratch_types=[
            pltpu.VMEM((4, 2, _CH), jnp.int32),
            pltpu.VMEM((2, _CH, _D), jnp.float32),
            pltpu.VMEM((_ZR, _D), jnp.float32),
            pltpu.VMEM_SHARED((_N, _D), jnp.float32),
        ] + [pltpu.SemaphoreType.DMA] * 8,
        compiler_params=pltpu.CompilerParams(use_tc_tiling_on_sc=False),
    )


def _make_cnt():
    mesh = plsc.VectorSubcoreMesh(core_axis_name="c", subcore_axis_name="s")
    return pl.kernel(
        _cnt_body,
        mesh=mesh,
        out_type=[jax.ShapeDtypeStruct((_NC, _N, 16), jnp.float32)],
        scratch_types=[
            pltpu.VMEM((_NCH, _CH), jnp.int32),
            pltpu.VMEM((_CH, 16), jnp.float32),
            pltpu.VMEM((_RPT + 16, 16), jnp.float32),
            pltpu.VMEM_SHARED((_N, 16), jnp.float32),
            pltpu.SemaphoreType.DMA,
        ],
        compiler_params=pltpu.CompilerParams(use_tc_tiling_on_sc=False),
    )


def _tc_body(relu, res, p_ref, c_ref, h_ref, wl_ref, wr_ref, b_ref, o_ref):
    p = p_ref[...]
    agg = p[0] + p[1]
    cc = c_ref[...]
    cnt = (cc[0] + cc[1])[:, 0:1]
    mean = agg / jnp.maximum(cnt, 1.0)
    hh = h_ref[...]
    dn = (((1,), (1,)), ((), ()))
    out = (lax.dot_general(mean, wl_ref[...], dn,
                           preferred_element_type=jnp.float32,
                           precision=lax.Precision.HIGHEST)
           + lax.dot_general(hh, wr_ref[...], dn,
                             preferred_element_type=jnp.float32,
                             precision=lax.Precision.HIGHEST)
           + b_ref[...])
    if res:
        out = out + hh
    if relu:
        out = jnp.maximum(out, 0.0)
    o_ref[...] = out


def _tc_call(relu, res, part, cntp, h, wl, wr, b):
    B = 1000
    return pl.pallas_call(
        functools.partial(_tc_body, relu, res),
        grid=(_N // B,),
        in_specs=[
            pl.BlockSpec((_NC, B, _D), lambda i: (0, i, 0)),
            pl.BlockSpec((_NC, B, 16), lambda i: (0, i, 0)),
            pl.BlockSpec((B, _D), lambda i: (i, 0)),
            pl.BlockSpec((_D, _D), lambda i: (0, 0)),
            pl.BlockSpec((_D, _D), lambda i: (0, 0)),
            pl.BlockSpec((1, _D), lambda i: (0, 0)),
        ],
        out_specs=pl.BlockSpec((B, _D), lambda i: (i, 0)),
        out_shape=jax.ShapeDtypeStruct((_N, _D), jnp.float32),
    )(part, cntp, h, wl, wr, b)


def kernel(x, edge_index, Wl0, Wr0, b0, Wl1, Wr1, b1, Wl2, Wr2, b2):
    src = edge_index[0].reshape(_NW, _NCH, _CH)
    dst = edge_index[1].reshape(_NW, _NCH, _CH)
    e3 = jnp.stack([src, dst], axis=2)

    sc = _make_sc()
    (cntp,) = _make_cnt()(dst)
    (part0,) = sc(x, e3)
    h1 = _tc_call(True, False, part0, cntp, x, Wl0, Wr0, b0.reshape(1, _D))
    (part1,) = sc(h1, e3)
    h2 = _tc_call(True, True, part1, cntp, h1, Wl1, Wr1, b1.reshape(1, _D))
    (part2,) = sc(h2, e3)
    return _tc_call(False, False, part2, cntp, h2, Wl2, Wr2, b2.reshape(1, _D))
